# initial kernel scaffold (unmeasured)
import os

import jax
import jax.numpy as jnp
from jax import lax
from jax.experimental import pallas as pl
from jax.experimental.pallas import tpu as pltpu

N_DEV = 32


def kernel(x, Win0, Wout0, Win1, Wout1, Win2, Wout2):
    b, d_shard = x.shape
    h_dim = Win0.shape[1]
    ch = b // N_DEV
    assert ch * N_DEV == b

    def body(x_ref, win0_ref, wout0_ref, win1_ref, wout1_ref, win2_ref,
             wout2_ref, out_ref, h_ref, rs_ref, hfull_ref,
             rs_send, rs_recv, ag_send, ag_recv):
        me = lax.axis_index("i")

        def peer(d):
            return lax.rem(me + d, N_DEV)

        def prev_peer(d):
            return lax.rem(me - d + N_DEV, N_DEV)

        barrier = pltpu.get_barrier_semaphore()
        for d in range(1, N_DEV):
            pl.semaphore_signal(barrier, inc=1, device_id=(peer(d),),
                                device_id_type=pl.DeviceIdType.MESH)
        pl.semaphore_wait(barrier, N_DEV - 1)

        x_val = x_ref[...]
        layers = ((win0_ref, wout0_ref), (win1_ref, wout1_ref),
                  (win2_ref, wout2_ref))
        for win_ref, wout_ref in layers:
            partial = jnp.dot(x_val, win_ref[...],
                              preferred_element_type=jnp.float32)
            h_ref[...] = partial

            rs_ref[pl.ds(ch * me, ch), :] = lax.dynamic_slice_in_dim(
                partial, ch * me, ch, axis=0)
            rs_sends = []
            for d in range(1, N_DEV):
                j = peer(d)
                rdma = pltpu.make_async_remote_copy(
                    src_ref=h_ref.at[pl.ds(ch * j, ch), :],
                    dst_ref=rs_ref.at[pl.ds(ch * me, ch), :],
                    send_sem=rs_send.at[d],
                    recv_sem=rs_recv.at[d],
                    device_id=(j,),
                    device_id_type=pl.DeviceIdType.MESH,
                )
                rdma.start()
                rs_sends.append(rdma)
            for d in range(1, N_DEV):
                k = prev_peer(d)
                recv = pltpu.make_async_remote_copy(
                    src_ref=rs_ref.at[pl.ds(ch * k, ch), :],
                    dst_ref=rs_ref.at[pl.ds(ch * k, ch), :],
                    send_sem=rs_send.at[d],
                    recv_sem=rs_recv.at[d],
                    device_id=(k,),
                    device_id_type=pl.DeviceIdType.MESH,
                )
                recv.wait_recv()
            for rdma in rs_sends:
                rdma.wait_send()

            rs_val = rs_ref[...]
            acc = rs_val[0:ch, :]
            for k in range(1, N_DEV):
                acc = acc + rs_val[ch * k:ch * (k + 1), :]
            hfull_ref[pl.ds(ch * me, ch), :] = jnp.maximum(acc, 0.0)

            ag_sends = []
            for d in range(1, N_DEV):
                j = peer(d)
                rdma = pltpu.make_async_remote_copy(
                    src_ref=hfull_ref.at[pl.ds(ch * me, ch), :],
                    dst_ref=hfull_ref.at[pl.ds(ch * me, ch), :],
                    send_sem=ag_send.at[d],
                    recv_sem=ag_recv.at[d],
                    device_id=(j,),
                    device_id_type=pl.DeviceIdType.MESH,
                )
                rdma.start()
                ag_sends.append(rdma)
            for d in range(1, N_DEV):
                k = prev_peer(d)
                recv = pltpu.make_async_remote_copy(
                    src_ref=hfull_ref.at[pl.ds(ch * k, ch), :],
                    dst_ref=hfull_ref.at[pl.ds(ch * k, ch), :],
                    send_sem=ag_send.at[d],
                    recv_sem=ag_recv.at[d],
                    device_id=(k,),
                    device_id_type=pl.DeviceIdType.MESH,
                )
                recv.wait_recv()
            for rdma in ag_sends:
                rdma.wait_send()

            x_val = jnp.dot(hfull_ref[...], wout_ref[...],
                            preferred_element_type=jnp.float32)

        out_ref[...] = x_val

    interpret = (pltpu.InterpretParams()
                 if os.environ.get("SCBAND_INTERPRET") else False)
    return pl.pallas_call(
        body,
        out_shape=jax.ShapeDtypeStruct((b, d_shard), jnp.float32),
        in_specs=[pl.BlockSpec(memory_space=pltpu.VMEM)] * 7,
        out_specs=pl.BlockSpec(memory_space=pltpu.VMEM),
        scratch_shapes=[
            pltpu.VMEM((b, h_dim), jnp.float32),
            pltpu.VMEM((b, h_dim), jnp.float32),
            pltpu.VMEM((b, h_dim), jnp.float32),
            pltpu.SemaphoreType.DMA((N_DEV,)),
            pltpu.SemaphoreType.DMA((N_DEV,)),
            pltpu.SemaphoreType.DMA((N_DEV,)),
            pltpu.SemaphoreType.DMA((N_DEV,)),
        ],
        compiler_params=pltpu.CompilerParams(collective_id=0),
        interpret=interpret,
    )(x, Win0, Wout0, Win1, Wout1, Win2, Wout2)


# baseline (device time: 81952 ns/iter reference)
import os

import jax
import jax.numpy as jnp
from jax import lax
from jax.experimental import pallas as pl
from jax.experimental.pallas import tpu as pltpu

N_DEV = 32


def kernel(x, Win0, Wout0, Win1, Wout1, Win2, Wout2):
    b, d_shard = x.shape
    h_dim = Win0.shape[1]
    ch = b // N_DEV
    assert ch * N_DEV == b

    def body(x_ref, win0_ref, wout0_ref, win1_ref, wout1_ref, win2_ref,
             wout2_ref, out_ref, h_ref, rs_ref, hfull_ref, red_ref,
             rs_send, rs_recv, ag_send, ag_recv, local_sem):
        me = lax.axis_index("i")

        def peer(d):
            return lax.rem(me + d, N_DEV)

        def prev_peer(d):
            return lax.rem(me - d + N_DEV, N_DEV)

        barrier = pltpu.get_barrier_semaphore()
        for d in range(1, N_DEV):
            pl.semaphore_signal(barrier, inc=1, device_id=(peer(d),),
                                device_id_type=pl.DeviceIdType.MESH)
        pl.semaphore_wait(barrier, N_DEV - 1)

        x_val = x_ref[...]
        layers = ((win0_ref, wout0_ref), (win1_ref, wout1_ref),
                  (win2_ref, wout2_ref))
        for win_ref, wout_ref in layers:
            partial = jnp.dot(x_val, win_ref[...],
                              preferred_element_type=jnp.float32)
            h_ref[...] = partial

            own = pltpu.make_async_copy(
                h_ref.at[pl.ds(ch * me, ch), :],
                rs_ref.at[pl.ds(ch * me, ch), :],
                local_sem,
            )
            own.start()
            rs_sends = []
            for d in range(1, N_DEV):
                j = peer(d)
                rdma = pltpu.make_async_remote_copy(
                    src_ref=h_ref.at[pl.ds(ch * j, ch), :],
                    dst_ref=rs_ref.at[pl.ds(ch * me, ch), :],
                    send_sem=rs_send.at[d],
                    recv_sem=rs_recv.at[d],
                    device_id=(j,),
                    device_id_type=pl.DeviceIdType.MESH,
                )
                rdma.start()
                rs_sends.append(rdma)
            for d in range(1, N_DEV):
                k = prev_peer(d)
                recv = pltpu.make_async_remote_copy(
                    src_ref=rs_ref.at[pl.ds(ch * k, ch), :],
                    dst_ref=rs_ref.at[pl.ds(ch * k, ch), :],
                    send_sem=rs_send.at[d],
                    recv_sem=rs_recv.at[d],
                    device_id=(k,),
                    device_id_type=pl.DeviceIdType.MESH,
                )
                recv.wait_recv()
            for rdma in rs_sends:
                rdma.wait_send()
            own.wait()

            rs_val = rs_ref[...]
            acc = rs_val[0:ch, :]
            for k in range(1, N_DEV):
                acc = acc + rs_val[ch * k:ch * (k + 1), :]
            red_ref[...] = jnp.maximum(acc, 0.0)
            mine = pltpu.make_async_copy(
                red_ref,
                hfull_ref.at[pl.ds(ch * me, ch), :],
                local_sem,
            )
            mine.start()

            ag_sends = []
            for d in range(1, N_DEV):
                j = peer(d)
                rdma = pltpu.make_async_remote_copy(
                    src_ref=red_ref,
                    dst_ref=hfull_ref.at[pl.ds(ch * me, ch), :],
                    send_sem=ag_send.at[d],
                    recv_sem=ag_recv.at[d],
                    device_id=(j,),
                    device_id_type=pl.DeviceIdType.MESH,
                )
                rdma.start()
                ag_sends.append(rdma)
            for d in range(1, N_DEV):
                k = prev_peer(d)
                recv = pltpu.make_async_remote_copy(
                    src_ref=hfull_ref.at[pl.ds(ch * k, ch), :],
                    dst_ref=hfull_ref.at[pl.ds(ch * k, ch), :],
                    send_sem=ag_send.at[d],
                    recv_sem=ag_recv.at[d],
                    device_id=(k,),
                    device_id_type=pl.DeviceIdType.MESH,
                )
                recv.wait_recv()
            for rdma in ag_sends:
                rdma.wait_send()
            mine.wait()

            x_val = jnp.dot(hfull_ref[...], wout_ref[...],
                            preferred_element_type=jnp.float32)

        out_ref[...] = x_val

    interpret = (pltpu.InterpretParams()
                 if os.environ.get("SCBAND_INTERPRET") else False)
    return pl.pallas_call(
        body,
        out_shape=jax.ShapeDtypeStruct((b, d_shard), jnp.float32),
        in_specs=[pl.BlockSpec(memory_space=pltpu.VMEM)] * 7,
        out_specs=pl.BlockSpec(memory_space=pltpu.VMEM),
        scratch_shapes=[
            pltpu.VMEM((b, h_dim), jnp.float32),
            pltpu.VMEM((b, h_dim), jnp.float32),
            pltpu.VMEM((b, h_dim), jnp.float32),
            pltpu.VMEM((ch, h_dim), jnp.float32),
            pltpu.SemaphoreType.DMA((N_DEV,)),
            pltpu.SemaphoreType.DMA((N_DEV,)),
            pltpu.SemaphoreType.DMA((N_DEV,)),
            pltpu.SemaphoreType.DMA((N_DEV,)),
            pltpu.SemaphoreType.DMA,
        ],
        compiler_params=pltpu.CompilerParams(
            collective_id=0, vmem_limit_bytes=60 * 1024 * 1024),
        interpret=interpret,
    )(x, Win0, Wout0, Win1, Wout1, Win2, Wout2)


# device time: 81519 ns/iter; 1.0053x vs baseline; 1.0053x over previous
import os

import jax
import jax.numpy as jnp
from jax import lax
from jax.experimental import pallas as pl
from jax.experimental.pallas import tpu as pltpu

N_DEV = 32


def kernel(x, Win0, Wout0, Win1, Wout1, Win2, Wout2):
    b, d_shard = x.shape
    h_dim = Win0.shape[1]
    ch = b // N_DEV
    assert ch * N_DEV == b

    def body(x_ref, win0_ref, wout0_ref, win1_ref, wout1_ref, win2_ref,
             wout2_ref, out_ref, h_ref, rs_ref, hfull_ref, red_ref,
             rs_send, rs_recv, ag_send, ag_recv, local_sem):
        me = lax.axis_index("i")

        def peer(d):
            return lax.rem(me + d, N_DEV)

        def prev_peer(d):
            return lax.rem(me - d + N_DEV, N_DEV)

        barrier = pltpu.get_barrier_semaphore()
        for d in range(1, N_DEV):
            pl.semaphore_signal(barrier, inc=1, device_id=(peer(d),),
                                device_id_type=pl.DeviceIdType.MESH)
        pl.semaphore_wait(barrier, N_DEV - 1)

        x_val = x_ref[...]
        layers = ((win0_ref, wout0_ref), (win1_ref, wout1_ref),
                  (win2_ref, wout2_ref))
        for win_ref, wout_ref in layers:
            partial = jnp.dot(x_val.astype(jnp.bfloat16),
                              win_ref[...].astype(jnp.bfloat16),
                              preferred_element_type=jnp.float32)
            h_ref[...] = partial

            own = pltpu.make_async_copy(
                h_ref.at[pl.ds(ch * me, ch), :],
                rs_ref.at[pl.ds(ch * me, ch), :],
                local_sem,
            )
            own.start()
            rs_sends = []
            for d in range(1, N_DEV):
                j = peer(d)
                rdma = pltpu.make_async_remote_copy(
                    src_ref=h_ref.at[pl.ds(ch * j, ch), :],
                    dst_ref=rs_ref.at[pl.ds(ch * me, ch), :],
                    send_sem=rs_send.at[d],
                    recv_sem=rs_recv.at[d],
                    device_id=(j,),
                    device_id_type=pl.DeviceIdType.MESH,
                )
                rdma.start()
                rs_sends.append(rdma)
            for d in range(1, N_DEV):
                k = prev_peer(d)
                recv = pltpu.make_async_remote_copy(
                    src_ref=rs_ref.at[pl.ds(ch * k, ch), :],
                    dst_ref=rs_ref.at[pl.ds(ch * k, ch), :],
                    send_sem=rs_send.at[d],
                    recv_sem=rs_recv.at[d],
                    device_id=(k,),
                    device_id_type=pl.DeviceIdType.MESH,
                )
                recv.wait_recv()
            for rdma in rs_sends:
                rdma.wait_send()
            own.wait()

            rs_val = rs_ref[...]
            acc = rs_val[0:ch, :]
            for k in range(1, N_DEV):
                acc = acc + rs_val[ch * k:ch * (k + 1), :]
            red_ref[...] = jnp.maximum(acc, 0.0)
            mine = pltpu.make_async_copy(
                red_ref,
                hfull_ref.at[pl.ds(ch * me, ch), :],
                local_sem,
            )
            mine.start()

            ag_sends = []
            for d in range(1, N_DEV):
                j = peer(d)
                rdma = pltpu.make_async_remote_copy(
                    src_ref=red_ref,
                    dst_ref=hfull_ref.at[pl.ds(ch * me, ch), :],
                    send_sem=ag_send.at[d],
                    recv_sem=ag_recv.at[d],
                    device_id=(j,),
                    device_id_type=pl.DeviceIdType.MESH,
                )
                rdma.start()
                ag_sends.append(rdma)
            for d in range(1, N_DEV):
                k = prev_peer(d)
                recv = pltpu.make_async_remote_copy(
                    src_ref=hfull_ref.at[pl.ds(ch * k, ch), :],
                    dst_ref=hfull_ref.at[pl.ds(ch * k, ch), :],
                    send_sem=ag_send.at[d],
                    recv_sem=ag_recv.at[d],
                    device_id=(k,),
                    device_id_type=pl.DeviceIdType.MESH,
                )
                recv.wait_recv()
            for rdma in ag_sends:
                rdma.wait_send()
            mine.wait()

            x_val = jnp.dot(hfull_ref[...].astype(jnp.bfloat16),
                            wout_ref[...].astype(jnp.bfloat16),
                            preferred_element_type=jnp.float32)

        out_ref[...] = x_val

    interpret = (pltpu.InterpretParams()
                 if os.environ.get("SCBAND_INTERPRET") else False)
    return pl.pallas_call(
        body,
        out_shape=jax.ShapeDtypeStruct((b, d_shard), jnp.float32),
        in_specs=[pl.BlockSpec(memory_space=pltpu.VMEM)] * 7,
        out_specs=pl.BlockSpec(memory_space=pltpu.VMEM),
        scratch_shapes=[
            pltpu.VMEM((b, h_dim), jnp.float32),
            pltpu.VMEM((b, h_dim), jnp.float32),
            pltpu.VMEM((b, h_dim), jnp.float32),
            pltpu.VMEM((ch, h_dim), jnp.float32),
            pltpu.SemaphoreType.DMA((N_DEV,)),
            pltpu.SemaphoreType.DMA((N_DEV,)),
            pltpu.SemaphoreType.DMA((N_DEV,)),
            pltpu.SemaphoreType.DMA((N_DEV,)),
            pltpu.SemaphoreType.DMA,
        ],
        compiler_params=pltpu.CompilerParams(
            collective_id=0, vmem_limit_bytes=60 * 1024 * 1024),
        interpret=interpret,
    )(x, Win0, Wout0, Win1, Wout1, Win2, Wout2)


# device time: 67516 ns/iter; 1.2138x vs baseline; 1.2074x over previous
import os

import jax
import jax.numpy as jnp
from jax import lax
from jax.experimental import pallas as pl
from jax.experimental.pallas import tpu as pltpu

N_DEV = 32


def kernel(x, Win0, Wout0, Win1, Wout1, Win2, Wout2):
    b, d_shard = x.shape
    h_dim = Win0.shape[1]
    ch = b // N_DEV
    assert ch * N_DEV == b

    def body(x_ref, win0_ref, wout0_ref, win1_ref, wout1_ref, win2_ref,
             wout2_ref, out_ref, h_ref, rs_ref, hfull_ref, red_ref,
             rs_send, rs_recv, ag_send, ag_recv, local_sem):
        me = lax.axis_index("i")

        def peer(d):
            return lax.rem(me + d, N_DEV)

        def prev_peer(d):
            return lax.rem(me - d + N_DEV, N_DEV)

        barrier = pltpu.get_barrier_semaphore()
        for d in range(1, N_DEV):
            pl.semaphore_signal(barrier, inc=1, device_id=(peer(d),),
                                device_id_type=pl.DeviceIdType.MESH)
        pl.semaphore_wait(barrier, N_DEV - 1)

        x_val = x_ref[...]
        layers = ((win0_ref, wout0_ref), (win1_ref, wout1_ref),
                  (win2_ref, wout2_ref))
        for win_ref, wout_ref in layers:
            partial = jnp.dot(x_val.astype(jnp.bfloat16),
                              win_ref[...].astype(jnp.bfloat16),
                              preferred_element_type=jnp.float32)
            h_ref[...] = partial.astype(jnp.bfloat16)

            own = pltpu.make_async_copy(
                h_ref.at[pl.ds(ch * me, ch), :],
                rs_ref.at[pl.ds(ch * me, ch), :],
                local_sem,
            )
            own.start()
            rs_sends = []
            for d in range(1, N_DEV):
                j = peer(d)
                rdma = pltpu.make_async_remote_copy(
                    src_ref=h_ref.at[pl.ds(ch * j, ch), :],
                    dst_ref=rs_ref.at[pl.ds(ch * me, ch), :],
                    send_sem=rs_send.at[d],
                    recv_sem=rs_recv.at[d],
                    device_id=(j,),
                    device_id_type=pl.DeviceIdType.MESH,
                )
                rdma.start()
                rs_sends.append(rdma)
            for d in range(1, N_DEV):
                k = prev_peer(d)
                recv = pltpu.make_async_remote_copy(
                    src_ref=rs_ref.at[pl.ds(ch * k, ch), :],
                    dst_ref=rs_ref.at[pl.ds(ch * k, ch), :],
                    send_sem=rs_send.at[d],
                    recv_sem=rs_recv.at[d],
                    device_id=(k,),
                    device_id_type=pl.DeviceIdType.MESH,
                )
                recv.wait_recv()
            for rdma in rs_sends:
                rdma.wait_send()
            own.wait()

            rs_val = rs_ref[...].astype(jnp.float32)
            acc = rs_val[0:ch, :]
            for k in range(1, N_DEV):
                acc = acc + rs_val[ch * k:ch * (k + 1), :]
            red_ref[...] = jnp.maximum(acc, 0.0).astype(jnp.bfloat16)
            mine = pltpu.make_async_copy(
                red_ref,
                hfull_ref.at[pl.ds(ch * me, ch), :],
                local_sem,
            )
            mine.start()

            ag_sends = []
            for d in range(1, N_DEV):
                j = peer(d)
                rdma = pltpu.make_async_remote_copy(
                    src_ref=red_ref,
                    dst_ref=hfull_ref.at[pl.ds(ch * me, ch), :],
                    send_sem=ag_send.at[d],
                    recv_sem=ag_recv.at[d],
                    device_id=(j,),
                    device_id_type=pl.DeviceIdType.MESH,
                )
                rdma.start()
                ag_sends.append(rdma)
            for d in range(1, N_DEV):
                k = prev_peer(d)
                recv = pltpu.make_async_remote_copy(
                    src_ref=hfull_ref.at[pl.ds(ch * k, ch), :],
                    dst_ref=hfull_ref.at[pl.ds(ch * k, ch), :],
                    send_sem=ag_send.at[d],
                    recv_sem=ag_recv.at[d],
                    device_id=(k,),
                    device_id_type=pl.DeviceIdType.MESH,
                )
                recv.wait_recv()
            for rdma in ag_sends:
                rdma.wait_send()
            mine.wait()

            x_val = jnp.dot(hfull_ref[...],
                            wout_ref[...].astype(jnp.bfloat16),
                            preferred_element_type=jnp.float32)

        out_ref[...] = x_val

    interpret = (pltpu.InterpretParams()
                 if os.environ.get("SCBAND_INTERPRET") else False)
    return pl.pallas_call(
        body,
        out_shape=jax.ShapeDtypeStruct((b, d_shard), jnp.float32),
        in_specs=[pl.BlockSpec(memory_space=pltpu.VMEM)] * 7,
        out_specs=pl.BlockSpec(memory_space=pltpu.VMEM),
        scratch_shapes=[
            pltpu.VMEM((b, h_dim), jnp.bfloat16),
            pltpu.VMEM((b, h_dim), jnp.bfloat16),
            pltpu.VMEM((b, h_dim), jnp.bfloat16),
            pltpu.VMEM((ch, h_dim), jnp.bfloat16),
            pltpu.SemaphoreType.DMA((N_DEV,)),
            pltpu.SemaphoreType.DMA((N_DEV,)),
            pltpu.SemaphoreType.DMA((N_DEV,)),
            pltpu.SemaphoreType.DMA((N_DEV,)),
            pltpu.SemaphoreType.DMA,
        ],
        compiler_params=pltpu.CompilerParams(
            collective_id=0, vmem_limit_bytes=60 * 1024 * 1024),
        interpret=interpret,
    )(x, Win0, Wout0, Win1, Wout1, Win2, Wout2)


# device time: 67264 ns/iter; 1.2184x vs baseline; 1.0037x over previous
import os

import jax
import jax.numpy as jnp
from jax import lax
from jax.experimental import pallas as pl
from jax.experimental.pallas import tpu as pltpu

N_DEV = 32


def kernel(x, Win0, Wout0, Win1, Wout1, Win2, Wout2):
    b, d_shard = x.shape
    h_dim = Win0.shape[1]
    ch = b // N_DEV
    hh = h_dim // 2
    assert ch * N_DEV == b

    def body(x_ref, win0_ref, wout0_ref, win1_ref, wout1_ref, win2_ref,
             wout2_ref, out_ref, h_ref, rsa_ref, rsb_ref, hfull_ref,
             reda_ref, redb_ref,
             rsa_send, rsa_recv, rsb_send, rsb_recv,
             aga_send, aga_recv, agb_send, agb_recv,
             local_a, local_b):
        me = lax.axis_index("i")

        def peer(d):
            return lax.rem(me + d, N_DEV)

        def prev_peer(d):
            return lax.rem(me - d + N_DEV, N_DEV)

        barrier = pltpu.get_barrier_semaphore()
        for d in range(1, N_DEV):
            pl.semaphore_signal(barrier, inc=1, device_id=(peer(d),),
                                device_id_type=pl.DeviceIdType.MESH)
        pl.semaphore_wait(barrier, N_DEV - 1)

        x_val = x_ref[...].astype(jnp.bfloat16)
        layers = ((win0_ref, wout0_ref), (win1_ref, wout1_ref),
                  (win2_ref, wout2_ref))
        for li, (win_ref, wout_ref) in enumerate(layers):
            h_ref[:, 0:hh] = jnp.dot(
                x_val, win_ref[:, 0:hh].astype(jnp.bfloat16),
                preferred_element_type=jnp.float32).astype(jnp.bfloat16)

            own_a = pltpu.make_async_copy(
                h_ref.at[pl.ds(ch * me, ch), 0:hh],
                rsa_ref.at[pl.ds(ch * me, ch), :], local_a)
            own_a.start()
            rsa_sends = []
            for d in range(1, N_DEV):
                j = peer(d)
                rdma = pltpu.make_async_remote_copy(
                    src_ref=h_ref.at[pl.ds(ch * j, ch), 0:hh],
                    dst_ref=rsa_ref.at[pl.ds(ch * me, ch), :],
                    send_sem=rsa_send.at[d], recv_sem=rsa_recv.at[d],
                    device_id=(j,), device_id_type=pl.DeviceIdType.MESH)
                rdma.start()
                rsa_sends.append(rdma)

            h_ref[:, hh:h_dim] = jnp.dot(
                x_val, win_ref[:, hh:h_dim].astype(jnp.bfloat16),
                preferred_element_type=jnp.float32).astype(jnp.bfloat16)

            own_b = pltpu.make_async_copy(
                h_ref.at[pl.ds(ch * me, ch), hh:h_dim],
                rsb_ref.at[pl.ds(ch * me, ch), :], local_b)
            own_b.start()
            rsb_sends = []
            for d in range(1, N_DEV):
                j = peer(d)
                rdma = pltpu.make_async_remote_copy(
                    src_ref=h_ref.at[pl.ds(ch * j, ch), hh:h_dim],
                    dst_ref=rsb_ref.at[pl.ds(ch * me, ch), :],
                    send_sem=rsb_send.at[d], recv_sem=rsb_recv.at[d],
                    device_id=(j,), device_id_type=pl.DeviceIdType.MESH)
                rdma.start()
                rsb_sends.append(rdma)

            for d in range(1, N_DEV):
                k = prev_peer(d)
                pltpu.make_async_remote_copy(
                    src_ref=rsa_ref.at[pl.ds(ch * k, ch), :],
                    dst_ref=rsa_ref.at[pl.ds(ch * k, ch), :],
                    send_sem=rsa_send.at[d], recv_sem=rsa_recv.at[d],
                    device_id=(k,), device_id_type=pl.DeviceIdType.MESH,
                ).wait_recv()
            own_a.wait()
            for rdma in rsa_sends:
                rdma.wait_send()

            rs_val = rsa_ref[...].astype(jnp.float32)
            acc = rs_val[0:ch, :]
            for k in range(1, N_DEV):
                acc = acc + rs_val[ch * k:ch * (k + 1), :]
            reda_ref[...] = jnp.maximum(acc, 0.0).astype(jnp.bfloat16)
            mine_a = pltpu.make_async_copy(
                reda_ref, hfull_ref.at[pl.ds(ch * me, ch), 0:hh], local_a)
            mine_a.start()
            aga_sends = []
            for d in range(1, N_DEV):
                rdma = pltpu.make_async_remote_copy(
                    src_ref=reda_ref,
                    dst_ref=hfull_ref.at[pl.ds(ch * me, ch), 0:hh],
                    send_sem=aga_send.at[d], recv_sem=aga_recv.at[d],
                    device_id=(peer(d),),
                    device_id_type=pl.DeviceIdType.MESH)
                rdma.start()
                aga_sends.append(rdma)

            for d in range(1, N_DEV):
                k = prev_peer(d)
                pltpu.make_async_remote_copy(
                    src_ref=rsb_ref.at[pl.ds(ch * k, ch), :],
                    dst_ref=rsb_ref.at[pl.ds(ch * k, ch), :],
                    send_sem=rsb_send.at[d], recv_sem=rsb_recv.at[d],
                    device_id=(k,), device_id_type=pl.DeviceIdType.MESH,
                ).wait_recv()
            own_b.wait()
            for rdma in rsb_sends:
                rdma.wait_send()
            rs_val = rsb_ref[...].astype(jnp.float32)
            acc = rs_val[0:ch, :]
            for k in range(1, N_DEV):
                acc = acc + rs_val[ch * k:ch * (k + 1), :]
            redb_ref[...] = jnp.maximum(acc, 0.0).astype(jnp.bfloat16)
            mine_b = pltpu.make_async_copy(
                redb_ref, hfull_ref.at[pl.ds(ch * me, ch), hh:h_dim],
                local_b)
            mine_b.start()
            agb_sends = []
            for d in range(1, N_DEV):
                rdma = pltpu.make_async_remote_copy(
                    src_ref=redb_ref,
                    dst_ref=hfull_ref.at[pl.ds(ch * me, ch), hh:h_dim],
                    send_sem=agb_send.at[d], recv_sem=agb_recv.at[d],
                    device_id=(peer(d),),
                    device_id_type=pl.DeviceIdType.MESH)
                rdma.start()
                agb_sends.append(rdma)

            for d in range(1, N_DEV):
                k = prev_peer(d)
                pltpu.make_async_remote_copy(
                    src_ref=reda_ref,
                    dst_ref=hfull_ref.at[pl.ds(ch * k, ch), 0:hh],
                    send_sem=aga_send.at[d], recv_sem=aga_recv.at[d],
                    device_id=(k,), device_id_type=pl.DeviceIdType.MESH,
                ).wait_recv()
            mine_a.wait()
            for rdma in aga_sends:
                rdma.wait_send()
            x_acc = jnp.dot(hfull_ref[:, 0:hh],
                            wout_ref[0:hh, :].astype(jnp.bfloat16),
                            preferred_element_type=jnp.float32)

            for d in range(1, N_DEV):
                k = prev_peer(d)
                pltpu.make_async_remote_copy(
                    src_ref=redb_ref,
                    dst_ref=hfull_ref.at[pl.ds(ch * k, ch), hh:h_dim],
                    send_sem=agb_send.at[d], recv_sem=agb_recv.at[d],
                    device_id=(k,), device_id_type=pl.DeviceIdType.MESH,
                ).wait_recv()
            mine_b.wait()
            for rdma in agb_sends:
                rdma.wait_send()
            x_f32 = x_acc + jnp.dot(hfull_ref[:, hh:h_dim],
                                    wout_ref[hh:h_dim, :].astype(jnp.bfloat16),
                                    preferred_element_type=jnp.float32)

            if li < 2:
                x_val = x_f32.astype(jnp.bfloat16)

        out_ref[...] = x_f32

    interpret = (pltpu.InterpretParams()
                 if os.environ.get("SCBAND_INTERPRET") else False)
    return pl.pallas_call(
        body,
        out_shape=jax.ShapeDtypeStruct((b, d_shard), jnp.float32),
        in_specs=[pl.BlockSpec(memory_space=pltpu.VMEM)] * 7,
        out_specs=pl.BlockSpec(memory_space=pltpu.VMEM),
        scratch_shapes=[
            pltpu.VMEM((b, h_dim), jnp.bfloat16),
            pltpu.VMEM((b, hh), jnp.bfloat16),
            pltpu.VMEM((b, hh), jnp.bfloat16),
            pltpu.VMEM((b, h_dim), jnp.bfloat16),
            pltpu.VMEM((ch, hh), jnp.bfloat16),
            pltpu.VMEM((ch, hh), jnp.bfloat16),
            pltpu.SemaphoreType.DMA((N_DEV,)),
            pltpu.SemaphoreType.DMA((N_DEV,)),
            pltpu.SemaphoreType.DMA((N_DEV,)),
            pltpu.SemaphoreType.DMA((N_DEV,)),
            pltpu.SemaphoreType.DMA((N_DEV,)),
            pltpu.SemaphoreType.DMA((N_DEV,)),
            pltpu.SemaphoreType.DMA((N_DEV,)),
            pltpu.SemaphoreType.DMA((N_DEV,)),
            pltpu.SemaphoreType.DMA,
            pltpu.SemaphoreType.DMA,
        ],
        compiler_params=pltpu.CompilerParams(
            collective_id=0, vmem_limit_bytes=60 * 1024 * 1024),
        interpret=interpret,
    )(x, Win0, Wout0, Win1, Wout1, Win2, Wout2)
